# baseline (speedup 1.0000x reference)
import jax
import jax.numpy as jnp
from jax import lax
from jax.experimental import pallas as pl
from jax.experimental.pallas import tpu as pltpu
from jax.experimental.pallas import tpu_sc as plsc

B = 16384
C = 20
D = 64
NW = 32
BPW = B // NW
NB = 32
NCHUNK = BPW // NB
RPC = NB * C
IDXW = 80
IDXR = RPC // IDXW
NG = RPC // 16


def _rsqrt(nsq):
    i = plsc.bitcast(nsq, jnp.int32)
    y = plsc.bitcast(jnp.int32(0x5F3759DF) - (i >> 1), jnp.float32)
    for _ in range(3):
        y = y * (1.5 - 0.5 * nsq * y * y)
    return y


def _scales_grp(rows, raw, scale_ref, inv, ptmp, r0, lanei):
    for i in range(16):
        r = r0 + i
        v0 = rows[r, pl.ds(0, 16)]
        v1 = rows[r, pl.ds(16, 16)]
        v2 = rows[r, pl.ds(32, 16)]
        v3 = rows[r, pl.ds(48, 16)]
        ptmp[i, pl.ds(0, 16)] = (v0 * v0 + v1 * v1) + (v2 * v2 + v3 * v3)
    accs = [jnp.zeros((16,), jnp.float32) for _ in range(4)]
    for l in range(16):
        accs[l % 4] = accs[l % 4] + plsc.load_gather(
            ptmp, [lanei, jnp.full((16,), l, jnp.int32)])
    nsq = (accs[0] + accs[1]) + (accs[2] + accs[3])
    s = jnp.where(nsq > 1.0, _rsqrt(nsq), 1.0)
    rflat = r0 + lanei
    iv = plsc.load_gather(raw, [rflat // C, rflat % C])
    s = jnp.where(iv == 0, 0.0, s) * inv
    scale_ref[pl.ds(r0, 16)] = s


def _sc_body(ctx_i, tgt_i, ctx_table, tgt_table, out,
             raw_c, raw_t, idx_c, idx_t, rows_c, rows_t, scale_c, scale_t,
             ce, pbuf, ptmp, ptmp2, simb, sem):
    wid = lax.axis_index("s") * 2 + lax.axis_index("c")

    def chunk(ch, carry):
        base = pl.multiple_of(wid * BPW + ch * NB, 8)
        pltpu.sync_copy(ctx_i.at[pl.ds(base, NB)], raw_c)
        pltpu.sync_copy(tgt_i.at[pl.ds(base, NB)], raw_t)

        def rbody(g, carry):
            rflat = g * 16 + lax.iota(jnp.int32, 16)
            q, r = rflat // C, rflat % C
            q8, r8 = rflat // IDXW, rflat % IDXW
            plsc.store_scatter(idx_c, [q8, r8], plsc.load_gather(raw_c, [q, r]))
            plsc.store_scatter(idx_t, [q8, r8], plsc.load_gather(raw_t, [q, r]))
            return carry

        lax.fori_loop(0, NG, rbody, 0)

        copies = []
        for j in range(IDXR):
            copies.append(pltpu.async_copy(
                ctx_table.at[idx_c.at[j]], rows_c.at[pl.ds(j * IDXW, IDXW)], sem))
            copies.append(pltpu.async_copy(
                tgt_table.at[idx_t.at[j]], rows_t.at[pl.ds(j * IDXW, IDXW)], sem))
        for cp in copies:
            cp.wait()

        def sbody(g, carry):
            r0 = g * 16
            lanei = lax.iota(jnp.int32, 16)
            _scales_grp(rows_c, raw_c, scale_c, 1.0 / C, ptmp, r0, lanei)
            _scales_grp(rows_t, raw_t, scale_t, 1.0, ptmp2, r0, lanei)
            return carry

        lax.fori_loop(0, NG, sbody, 0)

        def cbody(b, carry):
            accs = [jnp.zeros((16,), jnp.float32) for _ in range(8)]
            for c in range(C):
                r = b * C + c
                s = plsc.load_gather(scale_c, [jnp.full((16,), r, jnp.int32)])
                h = (c % 2) * 4
                for k in range(4):
                    accs[h + k] = accs[h + k] + s * rows_c[r, pl.ds(k * 16, 16)]
            for k in range(4):
                ce[b, pl.ds(k * 16, 16)] = accs[k] + accs[4 + k]
            return carry

        lax.fori_loop(0, NB, cbody, 0)

        def dbody(b, carry):
            cv = [ce[b, pl.ds(k * 16, 16)] for k in range(4)]
            for t in range(C):
                r = b * C + t
                p0 = cv[0] * rows_t[r, pl.ds(0, 16)]
                p1 = cv[1] * rows_t[r, pl.ds(16, 16)]
                p2 = cv[2] * rows_t[r, pl.ds(32, 16)]
                p3 = cv[3] * rows_t[r, pl.ds(48, 16)]
                pbuf[r, pl.ds(0, 16)] = (p0 + p1) + (p2 + p3)
            return carry

        lax.fori_loop(0, NB, dbody, 0)

        def fbody(g, carry):
            r0 = g * 16
            lanei = lax.iota(jnp.int32, 16)
            accs = [jnp.zeros((16,), jnp.float32) for _ in range(4)]
            for l in range(16):
                accs[l % 4] = accs[l % 4] + plsc.load_gather(
                    pbuf, [r0 + lanei, jnp.full((16,), l, jnp.int32)])
            acc = ((accs[0] + accs[1]) + (accs[2] + accs[3])) \
                * scale_t[pl.ds(r0, 16)]
            rflat = r0 + lanei
            plsc.store_scatter(simb, [rflat // C, rflat % C], acc)
            return carry

        lax.fori_loop(0, NG, fbody, 0)

        pltpu.sync_copy(simb, out.at[pl.ds(base, NB)])
        return carry

    lax.fori_loop(0, NCHUNK, chunk, 0)


@jax.jit
def _run(ctx_i, tgt_i, ctx_table, tgt_table):
    mesh = plsc.VectorSubcoreMesh(core_axis_name="c", subcore_axis_name="s")
    f = pl.kernel(
        _sc_body,
        mesh=mesh,
        compiler_params=pltpu.CompilerParams(use_tc_tiling_on_sc=False,
                                             needs_layout_passes=False),
        out_type=jax.ShapeDtypeStruct((B, C), jnp.float32),
        scratch_types=[
            pltpu.VMEM((NB, C), jnp.int32),
            pltpu.VMEM((NB, C), jnp.int32),
            pltpu.VMEM((IDXR, IDXW), jnp.int32),
            pltpu.VMEM((IDXR, IDXW), jnp.int32),
            pltpu.VMEM((RPC, D), jnp.float32),
            pltpu.VMEM((RPC, D), jnp.float32),
            pltpu.VMEM((RPC,), jnp.float32),
            pltpu.VMEM((RPC,), jnp.float32),
            pltpu.VMEM((NB, D), jnp.float32),
            pltpu.VMEM((RPC, 16), jnp.float32),
            pltpu.VMEM((16, 16), jnp.float32),
            pltpu.VMEM((16, 16), jnp.float32),
            pltpu.VMEM((NB, C), jnp.float32),
            pltpu.SemaphoreType.DMA,
        ],
    )
    return f(ctx_i, tgt_i, ctx_table, tgt_table)


def kernel(contexts, targets, ctx_table, tgt_table):
    return _run(contexts, targets, ctx_table, tgt_table)
